# baseline (speedup 1.0000x reference)
import jax
import jax.numpy as jnp
from jax import lax
from jax.experimental import pallas as pl
from jax.experimental.pallas import tpu as pltpu
from jax.experimental.pallas import tpu_sc as plsc

N = 10000
E = 320000
D = 128
NSUB = 16
CHUNK = 128
NIDX = 4
CHUNKS_PER_TILE = 160
EPT = CHUNKS_PER_TILE * CHUNK
EPAD = EPT * NSUB
ACC_ROWS = 10240
ROWS_PER_TILE = ACC_ROWS // NSUB
OUT_REM = N - (NSUB - 1) * ROWS_PER_TILE
PAD_DST = 10100


def _make_sc_agg(with_counts):
  outs = [jax.ShapeDtypeStruct((2, N, D), jnp.float32)]
  scratch = [
      pltpu.VMEM_SHARED((ACC_ROWS, D), jnp.float32),
      pltpu.VMEM((CHUNK, D), jnp.float32),
      pltpu.SemaphoreType.DMA,
      pltpu.SemaphoreType.DMA,
  ]
  scratch += [pltpu.VMEM((CHUNK,), jnp.int32) for _ in range(2 * NIDX)]
  scratch += [pltpu.SemaphoreType.DMA for _ in range(NIDX)]
  if with_counts:
    outs.append(jax.ShapeDtypeStruct((2 * NSUB * ACC_ROWS,), jnp.float32))
    scratch.append(pltpu.VMEM((ACC_ROWS,), jnp.float32))

  def body(xcat, srcs, dsts, *rest):
    if with_counts:
      agg_out, cnt_out = rest[0], rest[1]
      rest = rest[2:]
      hist = rest[-1]
      rest = rest[:-1]
    else:
      agg_out = rest[0]
      rest = rest[1:]
    acc, rows, gsem, ssem = rest[0], rest[1], rest[2], rest[3]
    sidx = rest[4:4 + NIDX]
    didx = rest[4 + NIDX:4 + 2 * NIDX]
    isem = rest[4 + 2 * NIDX:4 + 3 * NIDX]
    c = lax.axis_index("c")
    s = lax.axis_index("s")

    def zrow(r, _):
      for k in range(D // 16):
        rows[r, pl.ds(k * 16, 16)] = jnp.zeros((16,), jnp.float32)
      return 0
    lax.fori_loop(0, CHUNK, zrow, 0)
    for i in range(ROWS_PER_TILE // CHUNK):
      pltpu.sync_copy(rows.at[pl.ds(0, 128)],
                      acc.at[pl.ds(s * ROWS_PER_TILE + i * 128, 128)])
    if with_counts:
      def zh(i, _):
        hist[pl.ds(i * 16, 16)] = jnp.zeros((16,), jnp.float32)
        return 0
      lax.fori_loop(0, ACC_ROWS // 16, zh, 0)
    plsc.subcore_barrier()

    base0 = c * EPAD + s * EPT

    def idx_start(j, p):
      b = base0 + j * CHUNK
      pltpu.async_copy(srcs.at[pl.ds(b, CHUNK)], sidx[p], isem[p])
      pltpu.async_copy(dsts.at[pl.ds(b, CHUNK)], didx[p], isem[p])

    def idx_wait(p):
      pltpu.make_async_copy(srcs.at[pl.ds(0, CHUNK)], sidx[p], isem[p]).wait()
      pltpu.make_async_copy(dsts.at[pl.ds(0, CHUNK)], didx[p], isem[p]).wait()

    def chunk(j, p):
      idx_wait(p)
      pltpu.async_copy(xcat.at[sidx[p]], rows, gsem).wait()
      pltpu.async_copy(rows, acc.at[didx[p]], ssem, add=True)
      idx_start(j + 2, (p + 2) % NIDX)
      if with_counts:
        for k in range(CHUNK // 16):
          d = didx[p][pl.ds(k * 16, 16)]
          plsc.addupdate_scatter(hist, [d], jnp.ones((16,), jnp.float32))
      pltpu.make_async_copy(rows, acc.at[didx[p]], ssem).wait()

    idx_start(0, 0)
    idx_start(1, 1)

    def step(t, _):
      for p in range(NIDX):
        chunk(NIDX * t + p, p)
      return 0
    lax.fori_loop(0, CHUNKS_PER_TILE // NIDX, step, 0)
    idx_wait(0)
    idx_wait(1)
    plsc.subcore_barrier()

    o = s * ROWS_PER_TILE

    @pl.when(s < NSUB - 1)
    def _():
      pltpu.sync_copy(acc.at[pl.ds(o, ROWS_PER_TILE)],
                      agg_out.at[c, pl.ds(o, ROWS_PER_TILE)])

    @pl.when(s == NSUB - 1)
    def _():
      ol = (NSUB - 1) * ROWS_PER_TILE
      pltpu.sync_copy(acc.at[pl.ds(ol, OUT_REM)],
                      agg_out.at[c, pl.ds(ol, OUT_REM)])

    if with_counts:
      pltpu.sync_copy(
          hist, cnt_out.at[pl.ds((c * NSUB + s) * ACC_ROWS, ACC_ROWS)])

  mesh = plsc.VectorSubcoreMesh(core_axis_name="c", subcore_axis_name="s",
                                num_cores=2, num_subcores=NSUB)
  return pl.kernel(
      body, out_type=outs, mesh=mesh, scratch_types=scratch,
      compiler_params=pltpu.CompilerParams(needs_layout_passes=False))


_sc_cache = {}


def _sc_agg_kernel(with_counts):
  if with_counts not in _sc_cache:
    _sc_cache[with_counts] = _make_sc_agg(with_counts)
  return _sc_cache[with_counts]


def _lrelu(t):
  return jnp.where(t > 0, t, 0.01 * t)


def _matT(a, w_ref):
  return lax.dot_general(a, w_ref[...], (((1,), (1,)), ((), ())),
                         preferred_element_type=jnp.float32)


RB = 1000


def _dense1_body(agg, cnt, xs, wl1, b1, wr1, wl2, b2, wr2, out):
  cw = jnp.maximum(jnp.sum(cnt[0], axis=0), 1.0)
  cr = jnp.maximum(jnp.sum(cnt[1], axis=0), 1.0)
  m1 = _matT(agg[0] / cw, wl1) + _matT(xs[1], wr1) + b1[0:1, :]
  u1 = _matT(agg[1] / cr, wl2) + _matT(xs[0], wr2) + b2[0:1, :]
  out[0] = _lrelu(u1)
  out[1] = _lrelu(m1)


def _dense2_body(agg, cnt, h1, wl3, b3, wr3, u2, m2):
  cw = jnp.maximum(jnp.sum(cnt[0], axis=0), 1.0)
  cr = jnp.maximum(jnp.sum(cnt[1], axis=0), 1.0)
  m2[...] = _lrelu(_matT(agg[0] / cw, wl3) + _matT(h1[1], wr3) + b3[0:1, :])
  u2[...] = _lrelu(_matT(agg[1] / cr, wl3) + _matT(h1[0], wr3) + b3[0:1, :])


def _row_spec():
  return pl.BlockSpec((2, RB, D), lambda i: (0, i, 0))


def _cnt_spec():
  return pl.BlockSpec((2, NSUB, RB, 1), lambda i: (0, 0, i, 0))


def _w_spec():
  return pl.BlockSpec((D, D), lambda i: (0, 0))


def _b_spec():
  return pl.BlockSpec((8, D), lambda i: (0, 0))


_dense1 = pl.pallas_call(
    _dense1_body,
    grid=(N // RB,),
    in_specs=[_row_spec(), _cnt_spec(), _row_spec(),
              _w_spec(), _b_spec(), _w_spec(),
              _w_spec(), _b_spec(), _w_spec()],
    out_specs=_row_spec(),
    out_shape=jax.ShapeDtypeStruct((2, N, D), jnp.float32),
)

_dense2 = pl.pallas_call(
    _dense2_body,
    grid=(N // RB,),
    in_specs=[_row_spec(), _cnt_spec(), _row_spec(),
              _w_spec(), _b_spec(), _w_spec()],
    out_specs=[pl.BlockSpec((RB, D), lambda i: (i, 0)),
               pl.BlockSpec((RB, D), lambda i: (i, 0))],
    out_shape=[jax.ShapeDtypeStruct((N, D), jnp.float32),
               jax.ShapeDtypeStruct((N, D), jnp.float32)],
)


def kernel(x_user, x_movie, edge_index_watched, edge_index_rev_watched,
           W_l1, b_l1, W_r1, W_l2, b_l2, W_r2, W_l3, b_l3, W_r3):
  ei_w = edge_index_watched.astype(jnp.int32)
  ei_r = edge_index_rev_watched.astype(jnp.int32)
  pad = EPAD - E
  zpad = jnp.zeros((pad,), jnp.int32)
  dpad = jnp.full((pad,), PAD_DST, jnp.int32)
  slack = jnp.zeros((2 * CHUNK,), jnp.int32)
  srcs = jnp.concatenate([ei_w[0], zpad, ei_r[0] + N, zpad, slack])
  dsts = jnp.concatenate([ei_w[1], dpad, ei_r[1], dpad,
                          jnp.full((2 * CHUNK,), PAD_DST, jnp.int32)])

  xcat1 = jnp.concatenate([x_user, x_movie], axis=0)
  agg1, cnt = _sc_agg_kernel(True)(xcat1, srcs, dsts)
  cntp = cnt.reshape(2, NSUB, ACC_ROWS, 1)
  xs1 = xcat1.reshape(2, N, D)
  b1r = jnp.broadcast_to(b_l1[None, :], (8, D))
  b2r = jnp.broadcast_to(b_l2[None, :], (8, D))
  b3r = jnp.broadcast_to(b_l3[None, :], (8, D))
  h1 = _dense1(agg1, cntp, xs1, W_l1, b1r, W_r1, W_l2, b2r, W_r2)
  agg2, = _sc_agg_kernel(False)(h1.reshape(2 * N, D), srcs, dsts)
  u2, m2 = _dense2(agg2, cntp, h1, W_l3, b3r, W_r3)
  return (u2, m2)


# speedup vs baseline: 1.4515x; 1.4515x over previous
import jax
import jax.numpy as jnp
from jax import lax
from jax.experimental import pallas as pl
from jax.experimental.pallas import tpu as pltpu
from jax.experimental.pallas import tpu_sc as plsc

N = 10000
E = 320000
D = 128
NSUB = 16
CHUNK = 128
CHUNKS_PER_TILE = 157
EPT = CHUNKS_PER_TILE * CHUNK
EPAD = EPT * NSUB
ACC_ROWS = 10240
ROWS_PER_TILE = ACC_ROWS // NSUB
OUT_REM = N - (NSUB - 1) * ROWS_PER_TILE
PAD_DST = 10100


def _make_sc_agg(with_counts):
  outs = [jax.ShapeDtypeStruct((2, N, D), jnp.float32)]
  scratch = [
      pltpu.VMEM_SHARED((ACC_ROWS, D), jnp.float32),
      pltpu.VMEM((CHUNK,), jnp.int32),
      pltpu.VMEM((CHUNK,), jnp.int32),
      pltpu.VMEM((CHUNK, D), jnp.float32),
      pltpu.SemaphoreType.DMA,
      pltpu.SemaphoreType.DMA,
  ]
  if with_counts:
    outs.append(jax.ShapeDtypeStruct((2 * NSUB * ACC_ROWS,), jnp.float32))
    scratch.append(pltpu.VMEM((ACC_ROWS,), jnp.float32))

  def body(xcat, srcs, dsts, *rest):
    if with_counts:
      agg_out, cnt_out, acc, sidx, didx, rows, gsem, ssem, hist = rest
    else:
      agg_out, acc, sidx, didx, rows, gsem, ssem = rest
    c = lax.axis_index("c")
    s = lax.axis_index("s")

    def zrow(r, _):
      for k in range(D // 16):
        rows[r, pl.ds(k * 16, 16)] = jnp.zeros((16,), jnp.float32)
      return 0
    lax.fori_loop(0, CHUNK, zrow, 0)
    for i in range(ROWS_PER_TILE // CHUNK):
      pltpu.sync_copy(rows.at[pl.ds(0, 128)],
                      acc.at[pl.ds(s * ROWS_PER_TILE + i * 128, 128)])
    if with_counts:
      def zh(i, _):
        hist[pl.ds(i * 16, 16)] = jnp.zeros((16,), jnp.float32)
        return 0
      lax.fori_loop(0, ACC_ROWS // 16, zh, 0)
    plsc.subcore_barrier()

    base0 = c * EPAD + s * EPT

    def step(j, _):
      b = base0 + j * CHUNK
      pltpu.sync_copy(srcs.at[pl.ds(b, CHUNK)], sidx)
      pltpu.sync_copy(dsts.at[pl.ds(b, CHUNK)], didx)
      pltpu.async_copy(xcat.at[sidx], rows, gsem).wait()
      sc = pltpu.async_copy(rows, acc.at[didx], ssem, add=True)
      if with_counts:
        for k in range(CHUNK // 16):
          d = didx[pl.ds(k * 16, 16)]
          plsc.addupdate_scatter(hist, [d], jnp.ones((16,), jnp.float32))
      sc.wait()
      return 0
    lax.fori_loop(0, CHUNKS_PER_TILE, step, 0)
    plsc.subcore_barrier()

    o = s * ROWS_PER_TILE

    @pl.when(s < NSUB - 1)
    def _():
      pltpu.sync_copy(acc.at[pl.ds(o, ROWS_PER_TILE)],
                      agg_out.at[c, pl.ds(o, ROWS_PER_TILE)])

    @pl.when(s == NSUB - 1)
    def _():
      ol = (NSUB - 1) * ROWS_PER_TILE
      pltpu.sync_copy(acc.at[pl.ds(ol, OUT_REM)],
                      agg_out.at[c, pl.ds(ol, OUT_REM)])

    if with_counts:
      pltpu.sync_copy(
          hist, cnt_out.at[pl.ds((c * NSUB + s) * ACC_ROWS, ACC_ROWS)])

  mesh = plsc.VectorSubcoreMesh(core_axis_name="c", subcore_axis_name="s",
                                num_cores=2, num_subcores=NSUB)
  return pl.kernel(
      body, out_type=outs, mesh=mesh, scratch_types=scratch,
      compiler_params=pltpu.CompilerParams(needs_layout_passes=False))


_sc_cache = {}


def _sc_agg_kernel(with_counts):
  if with_counts not in _sc_cache:
    _sc_cache[with_counts] = _make_sc_agg(with_counts)
  return _sc_cache[with_counts]


def _lrelu(t):
  return jnp.where(t > 0, t, 0.01 * t)


def _matT(a, w_ref):
  return lax.dot_general(a, w_ref[...], (((1,), (1,)), ((), ())),
                         preferred_element_type=jnp.float32)


RB = 1000


def _dense1_body(agg, cnt, xs, wl1, b1, wr1, wl2, b2, wr2, out):
  cw = jnp.maximum(jnp.sum(cnt[0], axis=0), 1.0)
  cr = jnp.maximum(jnp.sum(cnt[1], axis=0), 1.0)
  m1 = _matT(agg[0] / cw, wl1) + _matT(xs[1], wr1) + b1[0:1, :]
  u1 = _matT(agg[1] / cr, wl2) + _matT(xs[0], wr2) + b2[0:1, :]
  out[0] = _lrelu(u1)
  out[1] = _lrelu(m1)


def _dense2_body(agg, cnt, h1, wl3, b3, wr3, u2, m2):
  cw = jnp.maximum(jnp.sum(cnt[0], axis=0), 1.0)
  cr = jnp.maximum(jnp.sum(cnt[1], axis=0), 1.0)
  m2[...] = _lrelu(_matT(agg[0] / cw, wl3) + _matT(h1[1], wr3) + b3[0:1, :])
  u2[...] = _lrelu(_matT(agg[1] / cr, wl3) + _matT(h1[0], wr3) + b3[0:1, :])


def _row_spec():
  return pl.BlockSpec((2, RB, D), lambda i: (0, i, 0))


def _cnt_spec():
  return pl.BlockSpec((2, NSUB, RB, 1), lambda i: (0, 0, i, 0))


def _w_spec():
  return pl.BlockSpec((D, D), lambda i: (0, 0))


def _b_spec():
  return pl.BlockSpec((8, D), lambda i: (0, 0))


_dense1 = pl.pallas_call(
    _dense1_body,
    grid=(N // RB,),
    in_specs=[_row_spec(), _cnt_spec(), _row_spec(),
              _w_spec(), _b_spec(), _w_spec(),
              _w_spec(), _b_spec(), _w_spec()],
    out_specs=_row_spec(),
    out_shape=jax.ShapeDtypeStruct((2, N, D), jnp.float32),
)

_dense2 = pl.pallas_call(
    _dense2_body,
    grid=(N // RB,),
    in_specs=[_row_spec(), _cnt_spec(), _row_spec(),
              _w_spec(), _b_spec(), _w_spec()],
    out_specs=[pl.BlockSpec((RB, D), lambda i: (i, 0)),
               pl.BlockSpec((RB, D), lambda i: (i, 0))],
    out_shape=[jax.ShapeDtypeStruct((N, D), jnp.float32),
               jax.ShapeDtypeStruct((N, D), jnp.float32)],
)


def kernel(x_user, x_movie, edge_index_watched, edge_index_rev_watched,
           W_l1, b_l1, W_r1, W_l2, b_l2, W_r2, W_l3, b_l3, W_r3):
  ei_w = edge_index_watched.astype(jnp.int32)
  ei_r = edge_index_rev_watched.astype(jnp.int32)
  pad = EPAD - E
  zpad = jnp.zeros((pad,), jnp.int32)
  dpad = N + (jnp.arange(pad, dtype=jnp.int32) % (ACC_ROWS - N))
  srcs = jnp.concatenate([ei_w[0], zpad, ei_r[0] + N, zpad])
  dsts = jnp.concatenate([ei_w[1], dpad, ei_r[1], dpad])

  xcat1 = jnp.concatenate([x_user, x_movie], axis=0)
  agg1, cnt = _sc_agg_kernel(True)(xcat1, srcs, dsts)
  cntp = cnt.reshape(2, NSUB, ACC_ROWS, 1)
  xs1 = xcat1.reshape(2, N, D)
  b1r = jnp.broadcast_to(b_l1[None, :], (8, D))
  b2r = jnp.broadcast_to(b_l2[None, :], (8, D))
  b3r = jnp.broadcast_to(b_l3[None, :], (8, D))
  h1 = _dense1(agg1, cntp, xs1, W_l1, b1r, W_r1, W_l2, b2r, W_r2)
  agg2, = _sc_agg_kernel(False)(h1.reshape(2 * N, D), srcs, dsts)
  u2, m2 = _dense2(agg2, cntp, h1, W_l3, b3r, W_r3)
  return (u2, m2)


# speedup vs baseline: 1.5862x; 1.0928x over previous
import jax
import jax.numpy as jnp
from jax import lax
from jax.experimental import pallas as pl
from jax.experimental.pallas import tpu as pltpu
from jax.experimental.pallas import tpu_sc as plsc

N = 10000
E = 320000
D = 128
NSUB = 16
CHUNK = 128
CHUNKS_PER_TILE = 157
EPT = CHUNKS_PER_TILE * CHUNK
EPAD = EPT * NSUB
ACC_ROWS = 10240
ROWS_PER_TILE = ACC_ROWS // NSUB
OUT_REM = N - (NSUB - 1) * ROWS_PER_TILE
PAD_DST = 10100


def _make_sc_agg(with_counts):
  outs = [jax.ShapeDtypeStruct((2, N, D), jnp.float32)]
  scratch = [
      pltpu.VMEM_SHARED((ACC_ROWS, D), jnp.float32),
      pltpu.VMEM((CHUNK,), jnp.int32),
      pltpu.VMEM((CHUNK,), jnp.int32),
      pltpu.VMEM((CHUNK, D), jnp.float32),
      pltpu.SemaphoreType.DMA,
      pltpu.SemaphoreType.DMA,
  ]
  if with_counts:
    outs.append(jax.ShapeDtypeStruct((2 * NSUB * ACC_ROWS,), jnp.float32))
    scratch.append(pltpu.VMEM((ACC_ROWS,), jnp.float32))

  def body(xcat, srcs, dsts, *rest):
    if with_counts:
      agg_out, cnt_out, acc, sidx, didx, rows, gsem, ssem, hist = rest
    else:
      agg_out, acc, sidx, didx, rows, gsem, ssem = rest
    c = lax.axis_index("c")
    s = lax.axis_index("s")

    def zrow(r, _):
      for k in range(D // 16):
        rows[r, pl.ds(k * 16, 16)] = jnp.zeros((16,), jnp.float32)
      return 0
    lax.fori_loop(0, CHUNK, zrow, 0)
    for i in range(ROWS_PER_TILE // CHUNK):
      pltpu.sync_copy(rows.at[pl.ds(0, 128)],
                      acc.at[pl.ds(s * ROWS_PER_TILE + i * 128, 128)])
    if with_counts:
      def zh(i, _):
        hist[pl.ds(i * 16, 16)] = jnp.zeros((16,), jnp.float32)
        return 0
      lax.fori_loop(0, ACC_ROWS // 16, zh, 0)
    plsc.subcore_barrier()

    base0 = c * EPAD + s * EPT

    def step(j, _):
      b = base0 + j * CHUNK
      i1 = pltpu.async_copy(srcs.at[pl.ds(b, CHUNK)], sidx, gsem)
      i2 = pltpu.async_copy(dsts.at[pl.ds(b, CHUNK)], didx, ssem)
      i1.wait()
      i2.wait()
      pltpu.async_copy(xcat.at[sidx], rows, gsem).wait()
      sc = pltpu.async_copy(rows, acc.at[didx], ssem, add=True)
      if with_counts:
        for k in range(CHUNK // 16):
          d = didx[pl.ds(k * 16, 16)]
          plsc.addupdate_scatter(hist, [d], jnp.ones((16,), jnp.float32))
      sc.wait()
      return 0
    lax.fori_loop(0, CHUNKS_PER_TILE, step, 0)
    plsc.subcore_barrier()

    o = s * ROWS_PER_TILE

    @pl.when(s < NSUB - 1)
    def _():
      pltpu.sync_copy(acc.at[pl.ds(o, ROWS_PER_TILE)],
                      agg_out.at[c, pl.ds(o, ROWS_PER_TILE)])

    @pl.when(s == NSUB - 1)
    def _():
      ol = (NSUB - 1) * ROWS_PER_TILE
      pltpu.sync_copy(acc.at[pl.ds(ol, OUT_REM)],
                      agg_out.at[c, pl.ds(ol, OUT_REM)])

    if with_counts:
      pltpu.sync_copy(
          hist, cnt_out.at[pl.ds((c * NSUB + s) * ACC_ROWS, ACC_ROWS)])

  mesh = plsc.VectorSubcoreMesh(core_axis_name="c", subcore_axis_name="s",
                                num_cores=2, num_subcores=NSUB)
  return pl.kernel(
      body, out_type=outs, mesh=mesh, scratch_types=scratch,
      compiler_params=pltpu.CompilerParams(needs_layout_passes=False))


_sc_cache = {}


def _sc_agg_kernel(with_counts):
  if with_counts not in _sc_cache:
    _sc_cache[with_counts] = _make_sc_agg(with_counts)
  return _sc_cache[with_counts]


def _lrelu(t):
  return jnp.where(t > 0, t, 0.01 * t)


def _matT(a, w_ref):
  return lax.dot_general(a, w_ref[...], (((1,), (1,)), ((), ())),
                         preferred_element_type=jnp.float32)


RB = 1000


def _dense1_body(agg, cnt, xs, wl1, b1, wr1, wl2, b2, wr2, out):
  cw = jnp.maximum(jnp.sum(cnt[0], axis=0), 1.0)
  cr = jnp.maximum(jnp.sum(cnt[1], axis=0), 1.0)
  m1 = _matT(agg[0] / cw, wl1) + _matT(xs[1], wr1) + b1[0:1, :]
  u1 = _matT(agg[1] / cr, wl2) + _matT(xs[0], wr2) + b2[0:1, :]
  out[0] = _lrelu(u1)
  out[1] = _lrelu(m1)


def _dense2_body(agg, cnt, h1, wl3, b3, wr3, u2, m2):
  cw = jnp.maximum(jnp.sum(cnt[0], axis=0), 1.0)
  cr = jnp.maximum(jnp.sum(cnt[1], axis=0), 1.0)
  m2[...] = _lrelu(_matT(agg[0] / cw, wl3) + _matT(h1[1], wr3) + b3[0:1, :])
  u2[...] = _lrelu(_matT(agg[1] / cr, wl3) + _matT(h1[0], wr3) + b3[0:1, :])


def _row_spec():
  return pl.BlockSpec((2, RB, D), lambda i: (0, i, 0))


def _cnt_spec():
  return pl.BlockSpec((2, NSUB, RB, 1), lambda i: (0, 0, i, 0))


def _w_spec():
  return pl.BlockSpec((D, D), lambda i: (0, 0))


def _b_spec():
  return pl.BlockSpec((8, D), lambda i: (0, 0))


_dense1 = pl.pallas_call(
    _dense1_body,
    grid=(N // RB,),
    in_specs=[_row_spec(), _cnt_spec(), _row_spec(),
              _w_spec(), _b_spec(), _w_spec(),
              _w_spec(), _b_spec(), _w_spec()],
    out_specs=_row_spec(),
    out_shape=jax.ShapeDtypeStruct((2, N, D), jnp.float32),
)

_dense2 = pl.pallas_call(
    _dense2_body,
    grid=(N // RB,),
    in_specs=[_row_spec(), _cnt_spec(), _row_spec(),
              _w_spec(), _b_spec(), _w_spec()],
    out_specs=[pl.BlockSpec((RB, D), lambda i: (i, 0)),
               pl.BlockSpec((RB, D), lambda i: (i, 0))],
    out_shape=[jax.ShapeDtypeStruct((N, D), jnp.float32),
               jax.ShapeDtypeStruct((N, D), jnp.float32)],
)


def kernel(x_user, x_movie, edge_index_watched, edge_index_rev_watched,
           W_l1, b_l1, W_r1, W_l2, b_l2, W_r2, W_l3, b_l3, W_r3):
  ei_w = edge_index_watched.astype(jnp.int32)
  ei_r = edge_index_rev_watched.astype(jnp.int32)
  pad = EPAD - E
  zpad = jnp.zeros((pad,), jnp.int32)
  dpad = N + (jnp.arange(pad, dtype=jnp.int32) % (ACC_ROWS - N))
  srcs = jnp.concatenate([ei_w[0], zpad, ei_r[0] + N, zpad])
  dsts = jnp.concatenate([ei_w[1], dpad, ei_r[1], dpad])

  xcat1 = jnp.concatenate([x_user, x_movie], axis=0)
  agg1, cnt = _sc_agg_kernel(True)(xcat1, srcs, dsts)
  cntp = cnt.reshape(2, NSUB, ACC_ROWS, 1)
  xs1 = xcat1.reshape(2, N, D)
  b1r = jnp.broadcast_to(b_l1[None, :], (8, D))
  b2r = jnp.broadcast_to(b_l2[None, :], (8, D))
  b3r = jnp.broadcast_to(b_l3[None, :], (8, D))
  h1 = _dense1(agg1, cntp, xs1, W_l1, b1r, W_r1, W_l2, b2r, W_r2)
  agg2, = _sc_agg_kernel(False)(h1.reshape(2 * N, D), srcs, dsts)
  u2, m2 = _dense2(agg2, cntp, h1, W_l3, b3r, W_r3)
  return (u2, m2)
